# baseline (speedup 1.0000x reference)
import functools

import jax
import jax.numpy as jnp
from jax import lax
from jax.experimental import pallas as pl
from jax.experimental.pallas import tpu as pltpu
from jax.experimental.pallas import tpu_sc as plsc

_LANES = 16
_NUM_WORKERS = 32
_CHUNK = 16
_DEPTH = 3
_TC_ROWS = 2048
_TC_BLOCK = 64


def _sc_embed_call(n_rows, n_sc, hidden):
    per_w = n_sc // _NUM_WORKERS
    n_chunks = per_w // _CHUNK
    mesh = plsc.VectorSubcoreMesh(core_axis_name="c", subcore_axis_name="s")

    row_bufs = [pltpu.VMEM((_CHUNK, hidden), jnp.float32)
                for _ in range(2 * _DEPTH)]
    sems = [pltpu.SemaphoreType.DMA for _ in range(3 * _DEPTH + 2)]

    @functools.partial(
        pl.kernel,
        mesh=mesh,
        out_type=jax.ShapeDtypeStruct((n_rows, hidden), jnp.float32),
        scratch_types=[
            pltpu.VMEM((per_w,), jnp.int32),
            pltpu.VMEM((per_w,), jnp.int32),
        ] + row_bufs + sems,
    )
    def sc_embed(tok_hbm, pos_hbm, ttab_hbm, ptab_hbm, out_hbm,
                 tidx_v, pidx_v, *scratch):
        wid = lax.axis_index("s") * 2 + lax.axis_index("c")
        base = wid * per_w
        i_t = pltpu.async_copy(tok_hbm.at[pl.ds(base, per_w)], tidx_v,
                               scratch[5 * _DEPTH])
        i_p = pltpu.async_copy(pos_hbm.at[pl.ds(base, per_w)], pidx_v,
                               scratch[5 * _DEPTH + 1])
        i_t.wait()
        i_p.wait()

        n_vec = hidden // _LANES
        bufs = [(scratch[2 * b], scratch[2 * b + 1],
                 scratch[2 * _DEPTH + 3 * b],
                 scratch[2 * _DEPTH + 3 * b + 1],
                 scratch[2 * _DEPTH + 3 * b + 2]) for b in range(_DEPTH)]
        gathers = [None] * _DEPTH
        stores = [None] * _DEPTH

        def start_gather(ci, b):
            off = ci * _CHUNK
            tb, pb, s_tg, s_pg, _ = bufs[b]
            if stores[b] is not None:
                stores[b].wait()
                stores[b] = None
            g_t = pltpu.async_copy(
                ttab_hbm.at[tidx_v.at[pl.ds(off, _CHUNK)]], tb, s_tg)
            g_p = pltpu.async_copy(
                ptab_hbm.at[pidx_v.at[pl.ds(off, _CHUNK)]], pb, s_pg)
            gathers[b] = (g_t, g_p)

        for ci in range(min(_DEPTH - 1, n_chunks)):
            start_gather(ci, ci % _DEPTH)
        for ci in range(n_chunks):
            cur = ci % _DEPTH
            if ci + _DEPTH - 1 < n_chunks:
                start_gather(ci + _DEPTH - 1, (ci + _DEPTH - 1) % _DEPTH)
            g_t, g_p = gathers[cur]
            g_t.wait()
            g_p.wait()
            tb, pb, _, _, s_st = bufs[cur]

            @plsc.parallel_loop(0, n_vec, 1, unroll=2)
            def add_body(j, tb=tb, pb=pb):
                sl = pl.ds(j * _LANES, _LANES)
                for r in range(_CHUNK):
                    plsc.addupdate(tb.at[r, sl], pb[r, sl])

            stores[cur] = pltpu.async_copy(
                tb, out_hbm.at[pl.ds(base + ci * _CHUNK, _CHUNK)], s_st)
        for b in range(_DEPTH):
            if stores[b] is not None:
                stores[b].wait()

    return sc_embed


def _tc_gather_body(tidx_s, pidx_s, ttab, ptab, out_blk,
                    tok_v, pos_v, sem_t, sem_p):
    nb = pl.num_programs(0)
    blk = pl.program_id(0)

    def start_dmas(slot, blk_idx):
        for r in range(_TC_BLOCK):
            ti = tidx_s[blk_idx * _TC_BLOCK + r]
            pi = pidx_s[blk_idx * _TC_BLOCK + r]
            pltpu.make_async_copy(
                ttab.at[pl.ds(ti, 1)], tok_v.at[slot, pl.ds(r, 1)],
                sem_t.at[slot]).start()
            pltpu.make_async_copy(
                ptab.at[pl.ds(pi, 1)], pos_v.at[slot, pl.ds(r, 1)],
                sem_p.at[slot]).start()

    cur = lax.rem(blk, 2)

    @pl.when(blk == 0)
    def _():
        start_dmas(0, 0)

    @pl.when(blk + 1 < nb)
    def _():
        start_dmas(lax.rem(blk + 1, 2), blk + 1)

    pltpu.make_async_copy(
        ttab.at[pl.ds(0, _TC_BLOCK)], tok_v.at[cur], sem_t.at[cur]).wait()
    pltpu.make_async_copy(
        ptab.at[pl.ds(0, _TC_BLOCK)], pos_v.at[cur], sem_p.at[cur]).wait()

    out_blk[...] = tok_v[cur] + pos_v[cur]


def _tc_gather_call(hidden):
    nb = _TC_ROWS // _TC_BLOCK
    grid_spec = pltpu.PrefetchScalarGridSpec(
        num_scalar_prefetch=2,
        grid=(nb,),
        in_specs=[
            pl.BlockSpec(memory_space=pl.ANY),
            pl.BlockSpec(memory_space=pl.ANY),
        ],
        out_specs=pl.BlockSpec((_TC_BLOCK, hidden), lambda b, *_: (b, 0)),
        scratch_shapes=[
            pltpu.VMEM((2, _TC_BLOCK, hidden), jnp.float32),
            pltpu.VMEM((2, _TC_BLOCK, hidden), jnp.float32),
            pltpu.SemaphoreType.DMA((2,)),
            pltpu.SemaphoreType.DMA((2,)),
        ],
    )
    return pl.pallas_call(
        _tc_gather_body,
        grid_spec=grid_spec,
        out_shape=jax.ShapeDtypeStruct((_TC_ROWS, hidden), jnp.float32),
    )


def kernel(token_ids, position_ids, token_table, pos_table):
    b, s = token_ids.shape
    _, hidden = token_table.shape
    n_rows = b * s
    n_sc = n_rows - _TC_ROWS
    tids = token_ids.reshape(n_rows).astype(jnp.int32)
    pids = position_ids.reshape(n_rows).astype(jnp.int32)
    out_sc = _sc_embed_call(n_rows, n_sc, hidden)(
        tids[:n_sc], pids[:n_sc], token_table, pos_table)
    out_tc = _tc_gather_call(hidden)(
        tids[n_sc:], pids[n_sc:], token_table, pos_table)
    out = lax.dynamic_update_slice(out_sc, out_tc, (n_sc, 0))
    return out.reshape(b, s, hidden)


# speedup vs baseline: 1.0549x; 1.0549x over previous
import functools

import jax
import jax.numpy as jnp
from jax import lax
from jax.experimental import pallas as pl
from jax.experimental.pallas import tpu as pltpu
from jax.experimental.pallas import tpu_sc as plsc

_LANES = 16
_NUM_WORKERS = 32
_CHUNK = 16
_DEPTH = 3


def _sc_embed_call(batch, seq, hidden):
    per_w = (batch * seq) // _NUM_WORKERS
    n_chunks = per_w // _CHUNK
    wpb = seq // per_w
    mesh = plsc.VectorSubcoreMesh(core_axis_name="c", subcore_axis_name="s")

    row_bufs = [pltpu.VMEM((_CHUNK, hidden), jnp.float32)
                for _ in range(2 * _DEPTH)]
    sems = [pltpu.SemaphoreType.DMA for _ in range(3 * _DEPTH + 2)]

    @functools.partial(
        pl.kernel,
        mesh=mesh,
        out_type=jax.ShapeDtypeStruct((batch, seq, hidden), jnp.float32),
        scratch_types=[
            pltpu.VMEM((per_w,), jnp.int32),
            pltpu.VMEM((per_w,), jnp.int32),
        ] + row_bufs + sems,
    )
    def sc_embed(tok_hbm, pos_hbm, ttab_hbm, ptab_hbm, out_hbm,
                 tidx_v, pidx_v, *scratch):
        wid = lax.axis_index("s") * 2 + lax.axis_index("c")
        b0 = wid // wpb
        col = (wid % wpb) * per_w
        i_t = pltpu.async_copy(tok_hbm.at[b0, pl.ds(col, per_w)], tidx_v,
                               scratch[5 * _DEPTH])
        i_p = pltpu.async_copy(pos_hbm.at[b0, pl.ds(col, per_w)], pidx_v,
                               scratch[5 * _DEPTH + 1])
        i_t.wait()
        i_p.wait()

        n_vec = hidden // _LANES
        bufs = [(scratch[2 * b], scratch[2 * b + 1],
                 scratch[2 * _DEPTH + 3 * b],
                 scratch[2 * _DEPTH + 3 * b + 1],
                 scratch[2 * _DEPTH + 3 * b + 2]) for b in range(_DEPTH)]
        gathers = [None] * _DEPTH
        stores = [None] * _DEPTH

        def start_gather(ci, b):
            off = ci * _CHUNK
            tb, pb, s_tg, s_pg, _ = bufs[b]
            if stores[b] is not None:
                stores[b].wait()
                stores[b] = None
            g_t = pltpu.async_copy(
                ttab_hbm.at[tidx_v.at[pl.ds(off, _CHUNK)]], tb, s_tg)
            g_p = pltpu.async_copy(
                ptab_hbm.at[pidx_v.at[pl.ds(off, _CHUNK)]], pb, s_pg)
            gathers[b] = (g_t, g_p)

        for ci in range(min(_DEPTH - 1, n_chunks)):
            start_gather(ci, ci % _DEPTH)
        for ci in range(n_chunks):
            cur = ci % _DEPTH
            if ci + _DEPTH - 1 < n_chunks:
                start_gather(ci + _DEPTH - 1, (ci + _DEPTH - 1) % _DEPTH)
            g_t, g_p = gathers[cur]
            g_t.wait()
            g_p.wait()
            tb, pb, _, _, s_st = bufs[cur]

            @plsc.parallel_loop(0, n_vec, 1, unroll=2)
            def add_body(j, tb=tb, pb=pb):
                sl = pl.ds(j * _LANES, _LANES)
                for r in range(_CHUNK):
                    plsc.addupdate(tb.at[r, sl], pb[r, sl])

            stores[cur] = pltpu.async_copy(
                tb, out_hbm.at[b0, pl.ds(col + ci * _CHUNK, _CHUNK)], s_st)
        for b in range(_DEPTH):
            if stores[b] is not None:
                stores[b].wait()

    return sc_embed


def kernel(token_ids, position_ids, token_table, pos_table):
    b, s = token_ids.shape
    _, hidden = token_table.shape
    tids = token_ids.astype(jnp.int32)
    pids = position_ids.astype(jnp.int32)
    return _sc_embed_call(b, s, hidden)(tids, pids, token_table, pos_table)
